# baseline (speedup 1.0000x reference)
import functools

import jax
import jax.numpy as jnp
import numpy as np
from jax import lax
from jax.experimental import pallas as pl
from jax.experimental.pallas import tpu as pltpu
from jax.experimental.pallas import tpu_sc as plsc

_NC = 2
_NS = 16
_L = 16
_NW = _NC * _NS

_N = 10000
_E = 320000
_D = 128
_DH = _D // 2
_NG = 50
_NGP = 64

_E_PER_W = _E // _NW
_EV_PER_W = _E_PER_W // _L

_E_PER_S = _E // _NS
_CH = 80
_NCH = _E_PER_S // _CH
_NBUF = 5
_NGRP = _NCH // _NBUF
_NB = _N // _CH
_NQ = -(-_NB // _NS)

_HV = _DH // _L


def _sc_mesh():
    return plsc.VectorSubcoreMesh(
        core_axis_name="c", subcore_axis_name="s",
        num_cores=_NC, num_subcores=_NS)


def _dist_body(px_h, py_h, pz_h, src_h, dst_h, d2_h,
               px_v, py_v, pz_v, src_v, dst_v, d2_v):
    c = lax.axis_index("c")
    s = lax.axis_index("s")
    w = s * _NC + c
    base = w * _E_PER_W
    pltpu.sync_copy(px_h, px_v)
    pltpu.sync_copy(py_h, py_v)
    pltpu.sync_copy(pz_h, pz_v)
    pltpu.sync_copy(src_h.at[pl.ds(base, _E_PER_W)], src_v)
    pltpu.sync_copy(dst_h.at[pl.ds(base, _E_PER_W)], dst_v)

    def step(i, carry):
        off = i * _L
        si = src_v[pl.ds(off, _L)]
        di = dst_v[pl.ds(off, _L)]
        ax = plsc.load_gather(px_v, [si])
        bx = plsc.load_gather(px_v, [di])
        ay = plsc.load_gather(py_v, [si])
        by = plsc.load_gather(py_v, [di])
        az = plsc.load_gather(pz_v, [si])
        bz = plsc.load_gather(pz_v, [di])
        dx = bx - ax
        dy = by - ay
        dz = bz - az
        d2_v[pl.ds(off, _L)] = dx * dx + dy * dy + dz * dz + 1e-12
        return carry

    lax.fori_loop(0, _EV_PER_W, step, 0)
    pltpu.sync_copy(d2_v, d2_h.at[pl.ds(base, _E_PER_W)])


def _run_dist(px, py, pz, src, dst):
    return pl.kernel(
        _dist_body,
        out_type=jax.ShapeDtypeStruct((_E,), jnp.float32),
        mesh=_sc_mesh(),
        scratch_types=[
            pltpu.VMEM((_N,), jnp.float32),
            pltpu.VMEM((_N,), jnp.float32),
            pltpu.VMEM((_N,), jnp.float32),
            pltpu.VMEM((_E_PER_W,), jnp.int32),
            pltpu.VMEM((_E_PER_W,), jnp.int32),
            pltpu.VMEM((_E_PER_W,), jnp.float32),
        ],
        compiler_params=pltpu.CompilerParams(needs_layout_passes=False),
    )(px, py, pz, src, dst)


_BE = 8192
_EP = 327680

_OFFSETS = np.zeros((1, _NGP), dtype=np.float32)
_OFFSETS[0, :_NG] = np.linspace(0.0, 5.0, _NG, dtype=np.float32)
_STEP = float(_OFFSETS[0, 1] - _OFFSETS[0, 0])
_COEFF = -0.5 / (_STEP * _STEP)

_PERM = np.zeros(_D, dtype=np.int32)
for _g in range(_D // 32):
    for _i in range(16):
        _PERM[32 * _g + 2 * _i] = 32 * _g + _i
        _PERM[32 * _g + 2 * _i + 1] = 32 * _g + 16 + _i


def _ef_body(d2_ref, off_ref, we_ref, be_ref, ef_ref):
    dist = jnp.sqrt(d2_ref[...]).reshape(_BE, 1)
    diff = dist - off_ref[...]
    g = jnp.exp(_COEFF * (diff * diff))
    ef = jnp.dot(g, we_ref[...], preferred_element_type=jnp.float32)
    ef_ref[...] = jnp.maximum(ef + be_ref[...], 0.0).astype(jnp.bfloat16)


def _run_edge_feat(d2, W_e_pad, b_e):
    d2p = jnp.pad(d2, (0, _EP - _E))
    return pl.pallas_call(
        _ef_body,
        grid=(_EP // _BE,),
        in_specs=[
            pl.BlockSpec((_BE,), lambda i: (i,)),
            pl.BlockSpec((1, _NGP), lambda i: (0, 0)),
            pl.BlockSpec((_NGP, _D), lambda i: (0, 0)),
            pl.BlockSpec((1, _D), lambda i: (0, 0)),
        ],
        out_specs=pl.BlockSpec((_BE, _D), lambda i: (i, 0)),
        out_shape=jax.ShapeDtypeStruct((_EP, _D), jnp.bfloat16),
    )(d2p, jnp.asarray(_OFFSETS), W_e_pad, b_e.reshape(1, _D))


def _msg_body(xs_h, srcr_h, dstr_h, ef_h, part_h,
              sidx, didx, rows, feat, msg,
              agg_sh, sem_i,
              sg0, sg1, sg2, sg3, sg4,
              se0, se1, se2, se3, se4):
    c = lax.axis_index("c")
    s = lax.axis_index("s")
    sgs = (sg0, sg1, sg2, sg3, sg4)
    ses = (se0, se1, se2, se3, se4)
    erow0 = s * _E_PER_S
    ecol = c * _DH

    def zstore(i, carry):
        msg[i // _HV, pl.ds((i % _HV) * _L, _L)] = jnp.zeros(
            (_L,), jnp.float32)
        return carry

    lax.fori_loop(0, _CH * _HV, zstore, 0)

    def zcopy(q, carry):
        b = q * _NS + s

        @pl.when(b < _NB)
        def _():
            pltpu.sync_copy(msg, agg_sh.at[pl.ds(b * _CH, _CH)])

        return carry

    lax.fori_loop(0, _NQ, zcopy, 0)
    plsc.subcore_barrier()

    def issue_idx(g, slot):
        pltpu.async_copy(srcr_h.at[c, s, pl.ds(g * _NBUF, _NBUF)],
                         sidx.at[slot], sem_i)
        pltpu.async_copy(dstr_h.at[s, pl.ds(g * _NBUF, _NBUF)],
                         didx.at[slot], sem_i)

    def drain_idx():
        pltpu.make_async_copy(srcr_h.at[c, s, pl.ds(0, _NBUF)],
                              sidx.at[0], sem_i).wait()
        pltpu.make_async_copy(dstr_h.at[s, pl.ds(0, _NBUF)],
                              didx.at[0], sem_i).wait()

    def issue_gather(slot, b, j):
        pltpu.async_copy(xs_h.at[sidx.at[slot, b]], rows.at[b], sgs[b])
        pltpu.async_copy(
            ef_h.at[pl.ds(erow0 + j * _CH, _CH), pl.ds(ecol, _DH)],
            feat.at[b], ses[b])

    def drain_gather(slot, b):
        pltpu.make_async_copy(xs_h.at[sidx.at[slot, b]],
                              rows.at[b], sgs[b]).wait()
        pltpu.make_async_copy(
            ef_h.at[pl.ds(erow0, _CH), pl.ds(ecol, _DH)],
            feat.at[b], ses[b]).wait()

    issue_idx(0, 0)
    drain_idx()
    for b in range(_NBUF):
        issue_gather(0, b, b)
    issue_idx(1, 1)

    def group(o, carry):
        par = lax.rem(o, 2)
        npar = 1 - par

        for b in range(_NBUF):
            j = o * _NBUF + b
            drain_gather(par, b)

            def mul(e, inner):
                for h in range(2):
                    x2 = rows[b, e, pl.ds(h * 2 * _L, 2 * _L)]
                    f2 = feat[b, e, pl.ds(h * 2 * _L, 2 * _L)]
                    xa, xb = plsc.unpack(
                        x2, format=plsc.PackFormat.INTERLEAVED)
                    fa, fb = plsc.unpack(
                        f2, format=plsc.PackFormat.INTERLEAVED)
                    msg[e, pl.ds(h * 2 * _L, _L)] = xa * fa
                    msg[e, pl.ds(h * 2 * _L + _L, _L)] = xb * fb
                return inner

            lax.fori_loop(0, _CH, mul, 0)
            pltpu.sync_copy(msg, agg_sh.at[didx.at[par, b]],
                            add=True)

            @pl.when(o + 1 < _NGRP)
            def _():
                if b == 0:
                    drain_idx()
                issue_gather(npar, b, j + _NBUF)

        @pl.when(o + 2 < _NGRP)
        def _():
            issue_idx(o + 2, par)

        return carry

    lax.fori_loop(0, _NGRP, group, 0)
    plsc.subcore_barrier()

    def wback(q, carry):
        b = q * _NS + s

        @pl.when(b < _NB)
        def _():
            pltpu.sync_copy(agg_sh.at[pl.ds(b * _CH, _CH)], msg)
            pltpu.sync_copy(msg,
                            part_h.at[b, slice(None), pl.ds(ecol, _DH)])

        return carry

    lax.fori_loop(0, _NQ, wback, 0)


def _run_messages(xs, src_r, dst_r, ef):
    return pl.kernel(
        _msg_body,
        out_type=jax.ShapeDtypeStruct((_NB, _CH, _D), jnp.float32),
        mesh=_sc_mesh(),
        scratch_types=[
            pltpu.VMEM((2, _NBUF, _CH), jnp.int32),
            pltpu.VMEM((2, _NBUF, _CH), jnp.int32),
            pltpu.VMEM((_NBUF, _CH, _DH), jnp.bfloat16),
            pltpu.VMEM((_NBUF, _CH, _DH), jnp.bfloat16),
            pltpu.VMEM((_CH, _DH), jnp.float32),
            pltpu.VMEM_SHARED((_N, _DH), jnp.float32),
            pltpu.SemaphoreType.DMA,
            pltpu.SemaphoreType.DMA,
            pltpu.SemaphoreType.DMA,
            pltpu.SemaphoreType.DMA,
            pltpu.SemaphoreType.DMA,
            pltpu.SemaphoreType.DMA,
            pltpu.SemaphoreType.DMA,
            pltpu.SemaphoreType.DMA,
            pltpu.SemaphoreType.DMA,
            pltpu.SemaphoreType.DMA,
            pltpu.SemaphoreType.DMA,
        ],
        compiler_params=pltpu.CompilerParams(
            needs_layout_passes=False, use_tc_tiling_on_sc=False),
    )(xs, src_r, dst_r, ef)


_BR = 1000


def _out_body(agg_ref, x_ref, wo_ref, ws_ref, bo_ref, o_ref):
    o_ref[...] = (
        jnp.dot(agg_ref[...], wo_ref[...], preferred_element_type=jnp.float32)
        + jnp.dot(x_ref[...], ws_ref[...], preferred_element_type=jnp.float32)
        + bo_ref[...])


def _run_out(agg, x, W_out, W_self, b_out):
    return pl.pallas_call(
        _out_body,
        grid=(_N // _BR,),
        in_specs=[
            pl.BlockSpec((_BR, _D), lambda i: (i, 0)),
            pl.BlockSpec((_BR, _D), lambda i: (i, 0)),
            pl.BlockSpec((_D, _D), lambda i: (0, 0)),
            pl.BlockSpec((_D, _D), lambda i: (0, 0)),
            pl.BlockSpec((1, _D), lambda i: (0, 0)),
        ],
        out_specs=pl.BlockSpec((_BR, _D), lambda i: (i, 0)),
        out_shape=jax.ShapeDtypeStruct((_N, _D), jnp.float32),
    )(agg, x, W_out, W_self, b_out.reshape(1, _D))


def kernel(x, pos, edge_index, W_e, b_e, W_self, W_out, b_out):
    src = edge_index[0]
    dst = edge_index[1]
    px = jnp.asarray(pos[:, 0], jnp.float32)
    py = jnp.asarray(pos[:, 1], jnp.float32)
    pz = jnp.asarray(pos[:, 2], jnp.float32)

    d2 = _run_dist(px, py, pz, src, dst)

    perm = jnp.asarray(_PERM)
    W_e_pad = jnp.zeros((_NGP, _D), jnp.float32).at[:_NG].set(W_e)
    ef = _run_edge_feat(d2, W_e_pad[:, perm], b_e[perm])

    xp = x[:, perm].astype(jnp.bfloat16)
    xs = jnp.concatenate([xp[:, :_DH], xp[:, _DH:]], axis=0)
    src2 = src.reshape(_NS, _NCH, _CH)
    src_r = jnp.stack([src2, src2 + _N])
    dst_r = dst.reshape(_NS, _NCH, _CH)
    part = _run_messages(xs, src_r, dst_r, ef)
    agg = part.reshape(_N, _D)

    return _run_out(agg, x, W_out, W_self, b_out)


# speedup vs baseline: 1.4668x; 1.4668x over previous
import functools

import jax
import jax.numpy as jnp
import numpy as np
from jax import lax
from jax.experimental import pallas as pl
from jax.experimental.pallas import tpu as pltpu
from jax.experimental.pallas import tpu_sc as plsc

_NC = 2
_NS = 16
_L = 16
_NW = _NC * _NS

_N = 10000
_E = 320000
_D = 128
_DH = _D // 2
_NG = 50
_NGP = 64

_E_PER_W = _E // _NW
_EV_PER_W = _E_PER_W // _L

_EH = 163840
_EP = 2 * _EH

_E_PER_S = _EH // _NS
_CH = 80
_NCH = _E_PER_S // _CH
_NBUF = 4
_NGRP = _NCH // _NBUF
_NB = _N // _CH
_NQ = -(-_NB // _NS)
_AGG_R = _N + 8

_HV = _DH // _L


def _sc_mesh():
    return plsc.VectorSubcoreMesh(
        core_axis_name="c", subcore_axis_name="s",
        num_cores=_NC, num_subcores=_NS)


def _dist_body(px_h, py_h, pz_h, src_h, dst_h, d2_h,
               px_v, py_v, pz_v, src_v, dst_v, d2_v):
    c = lax.axis_index("c")
    s = lax.axis_index("s")
    w = s * _NC + c
    base = w * _E_PER_W
    pltpu.sync_copy(px_h, px_v)
    pltpu.sync_copy(py_h, py_v)
    pltpu.sync_copy(pz_h, pz_v)
    pltpu.sync_copy(src_h.at[pl.ds(base, _E_PER_W)], src_v)
    pltpu.sync_copy(dst_h.at[pl.ds(base, _E_PER_W)], dst_v)

    def step(i, carry):
        off = i * _L
        si = src_v[pl.ds(off, _L)]
        di = dst_v[pl.ds(off, _L)]
        ax = plsc.load_gather(px_v, [si])
        bx = plsc.load_gather(px_v, [di])
        ay = plsc.load_gather(py_v, [si])
        by = plsc.load_gather(py_v, [di])
        az = plsc.load_gather(pz_v, [si])
        bz = plsc.load_gather(pz_v, [di])
        dx = bx - ax
        dy = by - ay
        dz = bz - az
        d2_v[pl.ds(off, _L)] = dx * dx + dy * dy + dz * dz + 1e-12
        return carry

    lax.fori_loop(0, _EV_PER_W, step, 0)
    pltpu.sync_copy(d2_v, d2_h.at[pl.ds(base, _E_PER_W)])


def _run_dist(px, py, pz, src, dst):
    return pl.kernel(
        _dist_body,
        out_type=jax.ShapeDtypeStruct((_E,), jnp.float32),
        mesh=_sc_mesh(),
        scratch_types=[
            pltpu.VMEM((_N,), jnp.float32),
            pltpu.VMEM((_N,), jnp.float32),
            pltpu.VMEM((_N,), jnp.float32),
            pltpu.VMEM((_E_PER_W,), jnp.int32),
            pltpu.VMEM((_E_PER_W,), jnp.int32),
            pltpu.VMEM((_E_PER_W,), jnp.float32),
        ],
        compiler_params=pltpu.CompilerParams(needs_layout_passes=False),
    )(px, py, pz, src, dst)


_BE = 8192
_NBLK = _EH // _BE

_OFFSETS = np.zeros((1, _NGP), dtype=np.float32)
_OFFSETS[0, :_NG] = np.linspace(0.0, 5.0, _NG, dtype=np.float32)
_STEP = float(_OFFSETS[0, 1] - _OFFSETS[0, 0])
_COEFF = -0.5 / (_STEP * _STEP)


def _ef_body(d2_ref, off_ref, we_ref, be_ref, ef_ref):
    dist = jnp.sqrt(d2_ref[...]).reshape(_BE, 1)
    diff = dist - off_ref[...]
    g = jnp.exp(_COEFF * (diff * diff))
    ef = jnp.dot(g, we_ref[...], preferred_element_type=jnp.float32)
    ef_ref[...] = jnp.maximum(ef + be_ref[...], 0.0)


def _run_edge_feat(d2p, W_e_pad, b_e, phase):
    return pl.pallas_call(
        _ef_body,
        grid=(_NBLK,),
        in_specs=[
            pl.BlockSpec((_BE,), lambda i: (i + phase * _NBLK,)),
            pl.BlockSpec((1, _NGP), lambda i: (0, 0)),
            pl.BlockSpec((_NGP, _D), lambda i: (0, 0)),
            pl.BlockSpec((1, _D), lambda i: (0, 0)),
        ],
        out_specs=pl.BlockSpec((_BE, _D), lambda i: (i, 0)),
        out_shape=jax.ShapeDtypeStruct((_EH, _D), jnp.float32),
    )(d2p, jnp.asarray(_OFFSETS), W_e_pad, b_e.reshape(1, _D))


def _msg_body(xs_h, srcr_h, dstr_h, ef_h, part_h,
              sidx, didx, rows, feat,
              agg_sh, sem_i,
              sg0, sg1, sg2, sg3,
              se0, se1, se2, se3):
    c = lax.axis_index("c")
    s = lax.axis_index("s")
    sgs = (sg0, sg1, sg2, sg3)
    ses = (se0, se1, se2, se3)
    erow0 = s * _E_PER_S
    ecol = c * _DH

    def zstore(i, carry):
        rows[0, i // _HV, pl.ds((i % _HV) * _L, _L)] = jnp.zeros(
            (_L,), jnp.float32)
        return carry

    lax.fori_loop(0, _CH * _HV, zstore, 0)

    def zcopy(q, carry):
        b = q * _NS + s

        @pl.when(b < _NB)
        def _():
            pltpu.sync_copy(rows.at[0], agg_sh.at[pl.ds(b * _CH, _CH)])

        return carry

    lax.fori_loop(0, _NQ, zcopy, 0)
    plsc.subcore_barrier()

    def issue_idx(g, slot):
        pltpu.async_copy(srcr_h.at[c, s, pl.ds(g * _NBUF, _NBUF)],
                         sidx.at[slot], sem_i)
        pltpu.async_copy(dstr_h.at[s, pl.ds(g * _NBUF, _NBUF)],
                         didx.at[slot], sem_i)

    def drain_idx():
        pltpu.make_async_copy(srcr_h.at[c, s, pl.ds(0, _NBUF)],
                              sidx.at[0], sem_i).wait()
        pltpu.make_async_copy(dstr_h.at[s, pl.ds(0, _NBUF)],
                              didx.at[0], sem_i).wait()

    def issue_gather(slot, b, j):
        pltpu.async_copy(xs_h.at[sidx.at[slot, b]], rows.at[b], sgs[b])
        pltpu.async_copy(
            ef_h.at[pl.ds(erow0 + j * _CH, _CH), pl.ds(ecol, _DH)],
            feat.at[b], ses[b])

    def drain_gather(slot, b):
        pltpu.make_async_copy(xs_h.at[sidx.at[slot, b]],
                              rows.at[b], sgs[b]).wait()
        pltpu.make_async_copy(
            ef_h.at[pl.ds(erow0, _CH), pl.ds(ecol, _DH)],
            feat.at[b], ses[b]).wait()

    issue_idx(0, 0)
    drain_idx()
    for b in range(_NBUF):
        issue_gather(0, b, b)
    issue_idx(1, 1)

    def group(o, carry):
        par = lax.rem(o, 2)
        npar = 1 - par

        for b in range(_NBUF):
            j = o * _NBUF + b
            drain_gather(par, b)

            def mul(e, inner):
                for k in range(_HV):
                    sl = pl.ds(k * _L, _L)
                    rows[b, e, sl] = rows[b, e, sl] * feat[b, e, sl]
                return inner

            lax.fori_loop(0, _CH, mul, 0)
            pltpu.sync_copy(rows.at[b], agg_sh.at[didx.at[par, b]],
                            add=True)

            @pl.when(o + 1 < _NGRP)
            def _():
                if b == 0:
                    drain_idx()
                issue_gather(npar, b, j + _NBUF)

        @pl.when(o + 2 < _NGRP)
        def _():
            issue_idx(o + 2, par)

        return carry

    lax.fori_loop(0, _NGRP, group, 0)
    plsc.subcore_barrier()

    def wback(q, carry):
        b = q * _NS + s

        @pl.when(b < _NB)
        def _():
            pltpu.sync_copy(agg_sh.at[pl.ds(b * _CH, _CH)], rows.at[0])
            pltpu.sync_copy(rows.at[0],
                            part_h.at[b, slice(None), pl.ds(ecol, _DH)])

        return carry

    lax.fori_loop(0, _NQ, wback, 0)


def _run_messages(xs, src_r, dst_r, ef):
    return pl.kernel(
        _msg_body,
        out_type=jax.ShapeDtypeStruct((_NB, _CH, _D), jnp.float32),
        mesh=_sc_mesh(),
        scratch_types=[
            pltpu.VMEM((2, _NBUF, _CH), jnp.int32),
            pltpu.VMEM((2, _NBUF, _CH), jnp.int32),
            pltpu.VMEM((_NBUF, _CH, _DH), jnp.float32),
            pltpu.VMEM((_NBUF, _CH, _DH), jnp.float32),
            pltpu.VMEM_SHARED((_AGG_R, _DH), jnp.float32),
            pltpu.SemaphoreType.DMA,
            pltpu.SemaphoreType.DMA,
            pltpu.SemaphoreType.DMA,
            pltpu.SemaphoreType.DMA,
            pltpu.SemaphoreType.DMA,
            pltpu.SemaphoreType.DMA,
            pltpu.SemaphoreType.DMA,
            pltpu.SemaphoreType.DMA,
            pltpu.SemaphoreType.DMA,
        ],
        compiler_params=pltpu.CompilerParams(
            needs_layout_passes=False, use_tc_tiling_on_sc=False),
    )(xs, src_r, dst_r, ef)


_BR = 1000


def _out_body(pa_ref, pb_ref, x_ref, wo_ref, ws_ref, bo_ref, o_ref):
    agg = pa_ref[...] + pb_ref[...]
    o_ref[...] = (
        jnp.dot(agg, wo_ref[...], preferred_element_type=jnp.float32)
        + jnp.dot(x_ref[...], ws_ref[...], preferred_element_type=jnp.float32)
        + bo_ref[...])


def _run_out(pa, pb, x, W_out, W_self, b_out):
    return pl.pallas_call(
        _out_body,
        grid=(_N // _BR,),
        in_specs=[
            pl.BlockSpec((_BR, _D), lambda i: (i, 0)),
            pl.BlockSpec((_BR, _D), lambda i: (i, 0)),
            pl.BlockSpec((_BR, _D), lambda i: (i, 0)),
            pl.BlockSpec((_D, _D), lambda i: (0, 0)),
            pl.BlockSpec((_D, _D), lambda i: (0, 0)),
            pl.BlockSpec((1, _D), lambda i: (0, 0)),
        ],
        out_specs=pl.BlockSpec((_BR, _D), lambda i: (i, 0)),
        out_shape=jax.ShapeDtypeStruct((_N, _D), jnp.float32),
    )(pa, pb, x, W_out, W_self, b_out.reshape(1, _D))


def kernel(x, pos, edge_index, W_e, b_e, W_self, W_out, b_out):
    src = edge_index[0]
    dst = edge_index[1]
    px = jnp.asarray(pos[:, 0], jnp.float32)
    py = jnp.asarray(pos[:, 1], jnp.float32)
    pz = jnp.asarray(pos[:, 2], jnp.float32)

    d2 = _run_dist(px, py, pz, src, dst)
    d2p = jnp.pad(d2, (0, _EP - _E))

    W_e_pad = jnp.zeros((_NGP, _D), jnp.float32).at[:_NG].set(W_e)
    ef_a = _run_edge_feat(d2p, W_e_pad, b_e, 0)
    ef_b = _run_edge_feat(d2p, W_e_pad, b_e, 1)

    xs = jnp.concatenate([x[:, :_DH], x[:, _DH:]], axis=0)
    srcp = jnp.pad(src, (0, _EP - _E)).reshape(2, _NS, _NCH, _CH)
    dstp = jnp.pad(dst, (0, _EP - _E),
                   constant_values=_N).reshape(2, _NS, _NCH, _CH)
    src_a = jnp.stack([srcp[0], srcp[0] + _N])
    src_b = jnp.stack([srcp[1], srcp[1] + _N])

    part_a = _run_messages(xs, src_a, dstp[0], ef_a)
    part_b = _run_messages(xs, src_b, dstp[1], ef_b)

    return _run_out(part_a.reshape(_N, _D), part_b.reshape(_N, _D),
                    x, W_out, W_self, b_out)


# speedup vs baseline: 2.3720x; 1.6172x over previous
import functools

import jax
import jax.numpy as jnp
import numpy as np
from jax import lax
from jax.experimental import pallas as pl
from jax.experimental.pallas import tpu as pltpu
from jax.experimental.pallas import tpu_sc as plsc

_NC = 2
_NS = 16
_L = 16
_NW = _NC * _NS

_N = 10000
_E = 320000
_D = 128
_DH = _D // 2
_NG = 50
_NGP = 64

_E_PER_W = _E // _NW
_EV_PER_W = _E_PER_W // _L

_E_PER_S = _E // _NS
_CH = 100
_NCH = _E_PER_S // _CH
_NBUF = 4
_NGRP = _NCH // _NBUF
_NB = _N // _CH
_NQ = -(-_NB // _NS)

_HV = _DH // _L


def _sc_mesh():
    return plsc.VectorSubcoreMesh(
        core_axis_name="c", subcore_axis_name="s",
        num_cores=_NC, num_subcores=_NS)


def _dist_body(px_h, py_h, pz_h, src_h, dst_h, d2_h,
               px_v, py_v, pz_v, src_v, dst_v, d2_v):
    c = lax.axis_index("c")
    s = lax.axis_index("s")
    w = s * _NC + c
    base = w * _E_PER_W
    pltpu.sync_copy(px_h, px_v)
    pltpu.sync_copy(py_h, py_v)
    pltpu.sync_copy(pz_h, pz_v)
    pltpu.sync_copy(src_h.at[pl.ds(base, _E_PER_W)], src_v)
    pltpu.sync_copy(dst_h.at[pl.ds(base, _E_PER_W)], dst_v)

    def step(i, carry):
        off = i * _L
        si = src_v[pl.ds(off, _L)]
        di = dst_v[pl.ds(off, _L)]
        ax = plsc.load_gather(px_v, [si])
        bx = plsc.load_gather(px_v, [di])
        ay = plsc.load_gather(py_v, [si])
        by = plsc.load_gather(py_v, [di])
        az = plsc.load_gather(pz_v, [si])
        bz = plsc.load_gather(pz_v, [di])
        dx = bx - ax
        dy = by - ay
        dz = bz - az
        d2_v[pl.ds(off, _L)] = dx * dx + dy * dy + dz * dz + 1e-12
        return carry

    lax.fori_loop(0, _EV_PER_W, step, 0)
    pltpu.sync_copy(d2_v, d2_h.at[pl.ds(base, _E_PER_W)])


def _run_dist(px, py, pz, src, dst):
    return pl.kernel(
        _dist_body,
        out_type=jax.ShapeDtypeStruct((_E,), jnp.float32),
        mesh=_sc_mesh(),
        scratch_types=[
            pltpu.VMEM((_N,), jnp.float32),
            pltpu.VMEM((_N,), jnp.float32),
            pltpu.VMEM((_N,), jnp.float32),
            pltpu.VMEM((_E_PER_W,), jnp.int32),
            pltpu.VMEM((_E_PER_W,), jnp.int32),
            pltpu.VMEM((_E_PER_W,), jnp.float32),
        ],
        compiler_params=pltpu.CompilerParams(needs_layout_passes=False),
    )(px, py, pz, src, dst)


_BE = 8192
_EP = 327680

_OFFSETS = np.zeros((1, _NGP), dtype=np.float32)
_OFFSETS[0, :_NG] = np.linspace(0.0, 5.0, _NG, dtype=np.float32)
_STEP = float(_OFFSETS[0, 1] - _OFFSETS[0, 0])
_COEFF = -0.5 / (_STEP * _STEP)


def _ef_body(d2_ref, off_ref, we_ref, be_ref, ef_ref):
    dist = jnp.sqrt(d2_ref[...]).reshape(_BE, 1)
    diff = dist - off_ref[...]
    g = jnp.exp(_COEFF * (diff * diff))
    ef = jnp.dot(g, we_ref[...], preferred_element_type=jnp.float32)
    ef_ref[...] = jnp.maximum(ef + be_ref[...], 0.0)


def _run_edge_feat(d2, W_e_pad, b_e):
    d2p = jnp.pad(d2, (0, _EP - _E))
    return pl.pallas_call(
        _ef_body,
        grid=(_EP // _BE,),
        in_specs=[
            pl.BlockSpec((_BE,), lambda i: (i,)),
            pl.BlockSpec((1, _NGP), lambda i: (0, 0)),
            pl.BlockSpec((_NGP, _D), lambda i: (0, 0)),
            pl.BlockSpec((1, _D), lambda i: (0, 0)),
        ],
        out_specs=pl.BlockSpec((_BE, _D), lambda i: (i, 0)),
        out_shape=jax.ShapeDtypeStruct((_EP, _D), jnp.float32),
    )(d2p, jnp.asarray(_OFFSETS), W_e_pad, b_e.reshape(1, _D))


def _msg_body(xs_h, srcr_h, dstr_h, ef_h, part_h,
              sidx, didx, rows, feat, msg,
              agg_sh, sem_i,
              sg0, sg1, sg2, sg3,
              se0, se1, se2, se3,
              ss0, ss1):
    c = lax.axis_index("c")
    s = lax.axis_index("s")
    sgs = (sg0, sg1, sg2, sg3)
    ses = (se0, se1, se2, se3)
    sss = (ss0, ss1)
    erow0 = s * _E_PER_S
    ecol = c * _DH

    def zstore(i, carry):
        msg[0, i // _HV, pl.ds((i % _HV) * _L, _L)] = jnp.zeros(
            (_L,), jnp.float32)
        return carry

    lax.fori_loop(0, _CH * _HV, zstore, 0)

    def zcopy(q, carry):
        b = q * _NS + s

        @pl.when(b < _NB)
        def _():
            pltpu.sync_copy(msg.at[0], agg_sh.at[pl.ds(b * _CH, _CH)])

        return carry

    lax.fori_loop(0, _NQ, zcopy, 0)
    plsc.subcore_barrier()

    def issue_idx(g, slot):
        pltpu.async_copy(srcr_h.at[c, s, pl.ds(g * _NBUF, _NBUF)],
                         sidx.at[slot], sem_i)
        pltpu.async_copy(dstr_h.at[s, pl.ds(g * _NBUF, _NBUF)],
                         didx.at[slot], sem_i)

    def drain_idx():
        pltpu.make_async_copy(srcr_h.at[c, s, pl.ds(0, _NBUF)],
                              sidx.at[0], sem_i).wait()
        pltpu.make_async_copy(dstr_h.at[s, pl.ds(0, _NBUF)],
                              didx.at[0], sem_i).wait()

    def issue_gather(slot, b, j):
        pltpu.async_copy(xs_h.at[sidx.at[slot, b]], rows.at[b], sgs[b])
        pltpu.async_copy(
            ef_h.at[pl.ds(erow0 + j * _CH, _CH), pl.ds(ecol, _DH)],
            feat.at[b], ses[b])

    def drain_gather(slot, b):
        pltpu.make_async_copy(xs_h.at[sidx.at[slot, b]],
                              rows.at[b], sgs[b]).wait()
        pltpu.make_async_copy(
            ef_h.at[pl.ds(erow0, _CH), pl.ds(ecol, _DH)],
            feat.at[b], ses[b]).wait()

    def drain_scatter(slot, m):
        pltpu.make_async_copy(msg.at[m], agg_sh.at[didx.at[slot, m]],
                              sss[m]).wait()

    issue_idx(0, 0)
    drain_idx()
    for b in range(_NBUF):
        issue_gather(0, b, b)
    issue_idx(1, 1)

    def group(o, carry):
        par = lax.rem(o, 3)
        npar = lax.rem(o + 1, 3)

        for b in range(_NBUF):
            j = o * _NBUF + b
            m = b % 2
            drain_gather(par, b)

            @pl.when(j >= 2)
            def _():
                drain_scatter(par, m)

            def mul(e, inner):
                for k in range(_HV):
                    sl = pl.ds(k * _L, _L)
                    msg[m, e, sl] = rows[b, e, sl] * feat[b, e, sl]
                return inner

            lax.fori_loop(0, _CH, mul, 0)
            pltpu.async_copy(msg.at[m], agg_sh.at[didx.at[par, b]],
                             sss[m], add=True)

            @pl.when(o + 1 < _NGRP)
            def _():
                if b == 0:
                    drain_idx()
                issue_gather(npar, b, j + _NBUF)

        @pl.when(o + 2 < _NGRP)
        def _():
            issue_idx(o + 2, lax.rem(o + 2, 3))

        return carry

    lax.fori_loop(0, _NGRP, group, 0)
    lpar = (_NGRP - 1) % 3
    drain_scatter(lpar, 0)
    drain_scatter(lpar, 1)
    plsc.subcore_barrier()

    def wback(q, carry):
        b = q * _NS + s

        @pl.when(b < _NB)
        def _():
            pltpu.sync_copy(agg_sh.at[pl.ds(b * _CH, _CH)], msg.at[0])
            pltpu.sync_copy(msg.at[0],
                            part_h.at[b, slice(None), pl.ds(ecol, _DH)])

        return carry

    lax.fori_loop(0, _NQ, wback, 0)


def _run_messages(xs, src_r, dst_r, ef):
    return pl.kernel(
        _msg_body,
        out_type=jax.ShapeDtypeStruct((_NB, _CH, _D), jnp.float32),
        mesh=_sc_mesh(),
        scratch_types=[
            pltpu.VMEM((3, _NBUF, _CH), jnp.int32),
            pltpu.VMEM((3, _NBUF, _CH), jnp.int32),
            pltpu.VMEM((_NBUF, _CH, _DH), jnp.float32),
            pltpu.VMEM((_NBUF, _CH, _DH), jnp.float32),
            pltpu.VMEM((2, _CH, _DH), jnp.float32),
            pltpu.VMEM_SHARED((_N, _DH), jnp.float32),
            pltpu.SemaphoreType.DMA,
            pltpu.SemaphoreType.DMA,
            pltpu.SemaphoreType.DMA,
            pltpu.SemaphoreType.DMA,
            pltpu.SemaphoreType.DMA,
            pltpu.SemaphoreType.DMA,
            pltpu.SemaphoreType.DMA,
            pltpu.SemaphoreType.DMA,
            pltpu.SemaphoreType.DMA,
            pltpu.SemaphoreType.DMA,
            pltpu.SemaphoreType.DMA,
        ],
        compiler_params=pltpu.CompilerParams(
            needs_layout_passes=False, use_tc_tiling_on_sc=False),
    )(xs, src_r, dst_r, ef)


_BR = 1000


def _out_body(agg_ref, x_ref, wo_ref, ws_ref, bo_ref, o_ref):
    o_ref[...] = (
        jnp.dot(agg_ref[...], wo_ref[...], preferred_element_type=jnp.float32)
        + jnp.dot(x_ref[...], ws_ref[...], preferred_element_type=jnp.float32)
        + bo_ref[...])


def _run_out(agg, x, W_out, W_self, b_out):
    return pl.pallas_call(
        _out_body,
        grid=(_N // _BR,),
        in_specs=[
            pl.BlockSpec((_BR, _D), lambda i: (i, 0)),
            pl.BlockSpec((_BR, _D), lambda i: (i, 0)),
            pl.BlockSpec((_D, _D), lambda i: (0, 0)),
            pl.BlockSpec((_D, _D), lambda i: (0, 0)),
            pl.BlockSpec((1, _D), lambda i: (0, 0)),
        ],
        out_specs=pl.BlockSpec((_BR, _D), lambda i: (i, 0)),
        out_shape=jax.ShapeDtypeStruct((_N, _D), jnp.float32),
    )(agg, x, W_out, W_self, b_out.reshape(1, _D))


def kernel(x, pos, edge_index, W_e, b_e, W_self, W_out, b_out):
    src = edge_index[0]
    dst = edge_index[1]
    px = jnp.asarray(pos[:, 0], jnp.float32)
    py = jnp.asarray(pos[:, 1], jnp.float32)
    pz = jnp.asarray(pos[:, 2], jnp.float32)

    d2 = _run_dist(px, py, pz, src, dst)

    W_e_pad = jnp.zeros((_NGP, _D), jnp.float32).at[:_NG].set(W_e)
    ef = _run_edge_feat(d2, W_e_pad, b_e)

    xs = jnp.concatenate([x[:, :_DH], x[:, _DH:]], axis=0)
    src2 = src.reshape(_NS, _NCH, _CH)
    src_r = jnp.stack([src2, src2 + _N])
    dst_r = dst.reshape(_NS, _NCH, _CH)
    part = _run_messages(xs, src_r, dst_r, ef)
    agg = part.reshape(_N, _D)

    return _run_out(agg, x, W_out, W_self, b_out)
